# baseline (speedup 1.0000x reference)
import functools
import math

import jax
import jax.numpy as jnp
from jax import lax
from jax.experimental import pallas as pl
from jax.experimental.pallas import tpu as pltpu
from jax.experimental.pallas import tpu_sc as plsc

VOCAB = 100000
D_MODEL = 1024
BATCH = 4
SEQ_LEN = 4096
N_ROWS = BATCH * SEQ_LEN
SCALE = math.sqrt(D_MODEL)

_info = plsc.get_sparse_core_info()
NUM_CORES = _info.num_cores
NUM_SUBCORES = _info.num_subcores
LANES = _info.num_lanes
NW = NUM_CORES * NUM_SUBCORES
S_PER_W = SEQ_LEN // NW
K = 8
GR = BATCH * K
N_GROUPS = S_PER_W // K
SETS = 3
VECS_PER_ROW = D_MODEL // LANES
IDX_ROWS_PER_B = SEQ_LEN // K


def _emb_body(x_ref, tok_ref, pos_ref, out_ref,
              idx3d, rowsbuf, posbuf,
              semg0, semg1, semg2, semp0, semp1, semp2,
              semw0, semw1, semw2):
    wid = lax.axis_index("s") * NUM_CORES + lax.axis_index("c")
    s0 = wid * S_PER_W

    pltpu.sync_copy(x_ref.at[pl.ds(wid * N_GROUPS, N_GROUPS)], idx3d)

    semg = (semg0, semg1, semg2)
    semp = (semp0, semp1, semp2)
    semw = (semw0, semw1, semw2)

    def gather_desc(g, par):
        return pltpu.make_async_copy(tok_ref.at[idx3d.at[g]],
                                     rowsbuf.at[par], semg[par])

    def pos_desc(g, par):
        return pltpu.make_async_copy(pos_ref.at[pl.ds(s0 + g * K, K)],
                                     posbuf.at[par], semp[par])

    def wb_desc(g, par, b):
        row0 = b * SEQ_LEN + s0 + g * K
        return pltpu.make_async_copy(rowsbuf.at[par, pl.ds(b * K, K)],
                                     out_ref.at[pl.ds(row0, K)], semw[par])

    def compute(par):
        def row_body(r, carry):
            for v in range(VECS_PER_ROW):
                sl = pl.ds(v * LANES, LANES)
                pv = posbuf[par, r, sl]
                for b in range(BATCH):
                    row = b * K + r
                    rowsbuf[par, row, sl] = rowsbuf[par, row, sl] * SCALE + pv
            return carry

        lax.fori_loop(0, K, row_body, 0, unroll=2)

    def start_group(g, par):
        pos_desc(g, par).start()
        gather_desc(g, par).start()

    def finish_group(g, par):
        pos_desc(g, par).wait()
        gather_desc(g, par).wait()
        compute(par)
        for b in range(BATCH):
            wb_desc(g, par, b).start()

    def drain_group(g, par):
        for b in range(BATCH):
            wb_desc(g, par, b).wait()

    def round_body(rp, carry):
        for j in range(SETS):
            g = SETS * rp + j
            @pl.when(rp >= 1)
            def _():
                drain_group(g - SETS, j)
            start_group(g, j)
            pj = (j - 1) % SETS
            if j == 0:
                @pl.when(rp >= 1)
                def _():
                    finish_group(g - 1, pj)
            else:
                finish_group(g - 1, pj)
        return carry

    n_loop = (N_GROUPS // SETS) * SETS
    lax.fori_loop(0, N_GROUPS // SETS, round_body, 0, unroll=1)

    for g in range(n_loop, N_GROUPS):
        par = g % SETS
        drain_group(g - SETS, par)
        start_group(g, par)
        finish_group(g - 1, (g - 1) % SETS)
    finish_group(N_GROUPS - 1, (N_GROUPS - 1) % SETS)
    for g in range(N_GROUPS - SETS, N_GROUPS):
        drain_group(g, g % SETS)


@jax.jit
def _emb_call(x2d, token_table, pos_table):
    mesh = plsc.VectorSubcoreMesh(core_axis_name="c", subcore_axis_name="s")
    f = functools.partial(
        pl.kernel,
        out_type=jax.ShapeDtypeStruct((N_ROWS, D_MODEL), jnp.float32),
        mesh=mesh,
        scratch_types=[
            pltpu.VMEM((N_GROUPS, GR), jnp.int32),
            pltpu.VMEM((SETS, GR, D_MODEL), jnp.float32),
            pltpu.VMEM((SETS, K, D_MODEL), jnp.float32),
            pltpu.SemaphoreType.DMA,
            pltpu.SemaphoreType.DMA,
            pltpu.SemaphoreType.DMA,
            pltpu.SemaphoreType.DMA,
            pltpu.SemaphoreType.DMA,
            pltpu.SemaphoreType.DMA,
            pltpu.SemaphoreType.DMA,
            pltpu.SemaphoreType.DMA,
            pltpu.SemaphoreType.DMA,
        ],
    )(_emb_body)
    return f(x2d, token_table, pos_table)


def kernel(x, token_table, pos_table):
    xp = (x.astype(jnp.int32)
          .reshape(BATCH, NW, N_GROUPS, K)
          .transpose(1, 2, 0, 3)
          .reshape(NW * N_GROUPS, GR))
    out = _emb_call(xp, token_table, pos_table)
    return out.reshape(BATCH, SEQ_LEN, D_MODEL)


# speedup vs baseline: 1.4620x; 1.4620x over previous
import functools
import math

import jax
import jax.numpy as jnp
from jax import lax
from jax.experimental import pallas as pl
from jax.experimental.pallas import tpu as pltpu
from jax.experimental.pallas import tpu_sc as plsc

VOCAB = 100000
D_MODEL = 1024
BATCH = 4
SEQ_LEN = 4096
N_ROWS = BATCH * SEQ_LEN
SCALE = math.sqrt(D_MODEL)

_info = plsc.get_sparse_core_info()
NUM_CORES = _info.num_cores
NUM_SUBCORES = _info.num_subcores
LANES = _info.num_lanes
NW = NUM_CORES * NUM_SUBCORES
S_PER_W = SEQ_LEN // NW
K = 8
GR = BATCH * K
N_GROUPS = S_PER_W // K
SETS = 3
VECS_PER_ROW = D_MODEL // LANES


def _emb_body(x_ref, tok_ref, pos_ref, out_ref,
              idx3d, rowsbuf, posbuf, semg, semp, semw):
    wid = lax.axis_index("s") * NUM_CORES + lax.axis_index("c")
    s0 = wid * S_PER_W

    pltpu.sync_copy(x_ref.at[pl.ds(wid * N_GROUPS, N_GROUPS)], idx3d)

    def gather_desc(g, par):
        return pltpu.make_async_copy(tok_ref.at[idx3d.at[g]],
                                     rowsbuf.at[par], semg.at[par])

    def pos_desc(g, par):
        return pltpu.make_async_copy(pos_ref.at[pl.ds(s0 + g * K, K)],
                                     posbuf.at[par], semp.at[par])

    def wb_desc(g, par, b):
        row0 = b * SEQ_LEN + s0 + g * K
        return pltpu.make_async_copy(rowsbuf.at[par, pl.ds(b * K, K)],
                                     out_ref.at[pl.ds(row0, K)], semw.at[par])

    def compute(par):
        def row_body(r, carry):
            for v in range(VECS_PER_ROW):
                sl = pl.ds(v * LANES, LANES)
                pv = posbuf[par, r, sl]
                for b in range(BATCH):
                    row = b * K + r
                    rowsbuf[par, row, sl] = rowsbuf[par, row, sl] * SCALE + pv
            return carry

        lax.fori_loop(0, K, row_body, 0, unroll=1)

    def step(g, carry):
        @pl.when(g < N_GROUPS)
        def _():
            par = lax.rem(g, SETS)

            @pl.when(g >= SETS)
            def _():
                for b in range(BATCH):
                    wb_desc(g - SETS, par, b).wait()

            pos_desc(g, par).start()
            gather_desc(g, par).start()

        @pl.when((g >= 1) & (g <= N_GROUPS))
        def _():
            gp = g - 1
            par = lax.rem(gp, SETS)
            pos_desc(gp, par).wait()
            gather_desc(gp, par).wait()
            compute(par)
            for b in range(BATCH):
                wb_desc(gp, par, b).start()

        @pl.when(g >= N_GROUPS)
        def _():
            gd = g - SETS
            par = lax.rem(gd, SETS)
            for b in range(BATCH):
                wb_desc(gd, par, b).wait()

        return carry

    lax.fori_loop(0, N_GROUPS + SETS, step, 0, unroll=1)


@jax.jit
def _emb_call(x2d, token_table, pos_table):
    mesh = plsc.VectorSubcoreMesh(core_axis_name="c", subcore_axis_name="s")
    f = functools.partial(
        pl.kernel,
        out_type=jax.ShapeDtypeStruct((N_ROWS, D_MODEL), jnp.float32),
        mesh=mesh,
        scratch_types=[
            pltpu.VMEM((N_GROUPS, GR), jnp.int32),
            pltpu.VMEM((SETS, GR, D_MODEL), jnp.float32),
            pltpu.VMEM((SETS, K, D_MODEL), jnp.float32),
            pltpu.SemaphoreType.DMA((SETS,)),
            pltpu.SemaphoreType.DMA((SETS,)),
            pltpu.SemaphoreType.DMA((SETS,)),
        ],
    )(_emb_body)
    return f(x2d, token_table, pos_table)


def kernel(x, token_table, pos_table):
    xp = (x.astype(jnp.int32)
          .reshape(BATCH, NW, N_GROUPS, K)
          .transpose(1, 2, 0, 3)
          .reshape(NW * N_GROUPS, GR))
    out = _emb_call(xp, token_table, pos_table)
    return out.reshape(BATCH, SEQ_LEN, D_MODEL)


# speedup vs baseline: 1.5842x; 1.0836x over previous
import functools
import math

import jax
import jax.numpy as jnp
from jax import lax
from jax.experimental import pallas as pl
from jax.experimental.pallas import tpu as pltpu
from jax.experimental.pallas import tpu_sc as plsc

VOCAB = 100000
D_MODEL = 1024
BATCH = 4
SEQ_LEN = 4096
N_ROWS = BATCH * SEQ_LEN
SCALE = math.sqrt(D_MODEL)

_info = plsc.get_sparse_core_info()
NUM_CORES = _info.num_cores
NUM_SUBCORES = _info.num_subcores
LANES = _info.num_lanes
NW = NUM_CORES * NUM_SUBCORES
S_PER_W = SEQ_LEN // NW
K = 8
GR = BATCH * K
N_GROUPS = S_PER_W // K
SETS = 3
VECS_PER_ROW = D_MODEL // LANES
IDX_ROWS_PER_B = SEQ_LEN // K


def _emb_body(x_ref, tok_ref, pos_ref, out_ref,
              idx3d, rowsbuf, posbuf,
              semg0, semg1, semg2, semp0, semp1, semp2,
              semw0, semw1, semw2):
    wid = lax.axis_index("s") * NUM_CORES + lax.axis_index("c")
    s0 = wid * S_PER_W

    pltpu.sync_copy(x_ref.at[pl.ds(wid * N_GROUPS, N_GROUPS)], idx3d)

    semg = (semg0, semg1, semg2)
    semp = (semp0, semp1, semp2)
    semw = (semw0, semw1, semw2)

    def gather_desc(g, par):
        return pltpu.make_async_copy(tok_ref.at[idx3d.at[g]],
                                     rowsbuf.at[par], semg[par])

    def pos_desc(g, par):
        return pltpu.make_async_copy(pos_ref.at[pl.ds(s0 + g * K, K)],
                                     posbuf.at[par], semp[par])

    def wb_desc(g, par, b):
        row0 = b * SEQ_LEN + s0 + g * K
        return pltpu.make_async_copy(rowsbuf.at[par, pl.ds(b * K, K)],
                                     out_ref.at[pl.ds(row0, K)], semw[par])

    def compute(par):
        def row_body(r, carry):
            for v in range(VECS_PER_ROW):
                sl = pl.ds(v * LANES, LANES)
                pv = posbuf[par, r, sl]
                for b in range(BATCH):
                    row = b * K + r
                    rowsbuf[par, row, sl] = rowsbuf[par, row, sl] * SCALE + pv
            return carry

        lax.fori_loop(0, K, row_body, 0, unroll=1)

    def start_group(g, par):
        pos_desc(g, par).start()
        gather_desc(g, par).start()

    def finish_group(g, par):
        pos_desc(g, par).wait()
        gather_desc(g, par).wait()
        compute(par)
        for b in range(BATCH):
            wb_desc(g, par, b).start()

    def drain_group(g, par):
        for b in range(BATCH):
            wb_desc(g, par, b).wait()

    def round_body(rp, carry):
        for j in range(SETS):
            g = SETS * rp + j
            @pl.when(rp >= 1)
            def _():
                drain_group(g - SETS, j)
            start_group(g, j)
            pj = (j - 1) % SETS
            if j == 0:
                @pl.when(rp >= 1)
                def _():
                    finish_group(g - 1, pj)
            else:
                finish_group(g - 1, pj)
        return carry

    n_loop = (N_GROUPS // SETS) * SETS
    lax.fori_loop(0, N_GROUPS // SETS, round_body, 0, unroll=1)

    for g in range(n_loop, N_GROUPS):
        par = g % SETS
        drain_group(g - SETS, par)
        start_group(g, par)
        finish_group(g - 1, (g - 1) % SETS)
    finish_group(N_GROUPS - 1, (N_GROUPS - 1) % SETS)
    for g in range(N_GROUPS - SETS, N_GROUPS):
        drain_group(g, g % SETS)


@jax.jit
def _emb_call(x2d, token_table, pos_table):
    mesh = plsc.VectorSubcoreMesh(core_axis_name="c", subcore_axis_name="s")
    f = functools.partial(
        pl.kernel,
        out_type=jax.ShapeDtypeStruct((N_ROWS, D_MODEL), jnp.float32),
        mesh=mesh,
        scratch_types=[
            pltpu.VMEM((N_GROUPS, GR), jnp.int32),
            pltpu.VMEM((SETS, GR, D_MODEL), jnp.float32),
            pltpu.VMEM((SETS, K, D_MODEL), jnp.float32),
            pltpu.SemaphoreType.DMA,
            pltpu.SemaphoreType.DMA,
            pltpu.SemaphoreType.DMA,
            pltpu.SemaphoreType.DMA,
            pltpu.SemaphoreType.DMA,
            pltpu.SemaphoreType.DMA,
            pltpu.SemaphoreType.DMA,
            pltpu.SemaphoreType.DMA,
            pltpu.SemaphoreType.DMA,
        ],
    )(_emb_body)
    return f(x2d, token_table, pos_table)


def kernel(x, token_table, pos_table):
    xp = (x.astype(jnp.int32)
          .reshape(BATCH, NW, N_GROUPS, K)
          .transpose(1, 2, 0, 3)
          .reshape(NW * N_GROUPS, GR))
    out = _emb_call(xp, token_table, pos_table)
    return out.reshape(BATCH, SEQ_LEN, D_MODEL)
